# baseline (speedup 1.0000x reference)
import functools

import jax
import jax.numpy as jnp
from jax import lax
from jax.experimental import pallas as pl
from jax.experimental.pallas import tpu as pltpu
from jax.experimental.pallas import tpu_sc as plsc

EMBED = 64
NC = 2
NS = 16
NW = NC * NS
NB = 8
NBUF = 2


@functools.partial(jax.jit, static_argnames=("batch", "hist"))
def _sc_gather(table, locations, batch, hist):
    b_per_w = batch // NW
    n_steps = b_per_w // NB
    n_outer = n_steps // NBUF
    mesh = plsc.VectorSubcoreMesh(core_axis_name="c", subcore_axis_name="s")

    @functools.partial(
        pl.kernel,
        mesh=mesh,
        out_type=jax.ShapeDtypeStruct((batch, hist, EMBED), jnp.float32),
        scratch_types=[
            pltpu.VMEM((b_per_w, hist), jnp.int32),
            *[pltpu.VMEM((NB, hist, EMBED), jnp.float32) for _ in range(NBUF)],
            pltpu.SemaphoreType.DMA,
            pltpu.SemaphoreType.DMA,
        ],
        compiler_params=pltpu.CompilerParams(use_tc_tiling_on_sc=False),
    )
    def k(table_hbm, idx_hbm, out_hbm, idx_v, *bufs_and_sems):
        bufs = bufs_and_sems[:NBUF]
        sem_g, sem_w = bufs_and_sems[NBUF:]
        wid = lax.axis_index("s") * NC + lax.axis_index("c")
        base = wid * b_per_w
        pltpu.sync_copy(idx_hbm.at[pl.ds(base, b_per_w)], idx_v)

        def wait_one_write():
            pltpu.make_async_copy(
                bufs[0], out_hbm.at[pl.ds(base, NB)], sem_w
            ).wait()

        def outer(g, _):
            descs = []
            for b in range(NBUF):
                t = g * NBUF + b

                @pl.when(g >= 1)
                def _():
                    wait_one_write()

                for j in range(NB):
                    desc = pltpu.make_async_copy(
                        table_hbm.at[idx_v.at[t * NB + j]],
                        bufs[b].at[j],
                        sem_g,
                    )
                    desc.start()
                    descs.append(desc)
            for b in range(NBUF):
                t = g * NBUF + b
                for j in range(NB):
                    descs[b * NB + j].wait()
                pltpu.make_async_copy(
                    bufs[b], out_hbm.at[pl.ds(base + t * NB, NB)], sem_w
                ).start()
            return 0

        lax.fori_loop(0, n_outer, outer, 0)
        for _ in range(NBUF):
            wait_one_write()

    return k(table, locations)


def kernel(locations, table):
    batch, hist = locations.shape
    out = _sc_gather(table, locations, batch, hist)
    return out * (1.0 + 0.0 * table[0, 0])


# speedup vs baseline: 1.2928x; 1.2928x over previous
import functools

import jax
import jax.numpy as jnp
from jax import lax
from jax.experimental import pallas as pl
from jax.experimental.pallas import tpu as pltpu
from jax.experimental.pallas import tpu_sc as plsc

EMBED = 64
NC = 2
NS = 16
NW = NC * NS
NB = 8
NBUF = 2


@functools.partial(jax.jit, static_argnames=("batch", "hist"))
def _sc_gather(table, locations, batch, hist):
    b_per_w = batch // NW
    n_steps = b_per_w // NB
    n_outer = n_steps // NBUF
    mesh = plsc.VectorSubcoreMesh(core_axis_name="c", subcore_axis_name="s")

    @functools.partial(
        pl.kernel,
        mesh=mesh,
        out_type=jax.ShapeDtypeStruct((batch, hist, EMBED), jnp.float32),
        scratch_types=[
            pltpu.VMEM((b_per_w, hist), jnp.int32),
            *[pltpu.VMEM((NB, hist, EMBED), jnp.float32) for _ in range(NBUF)],
            pltpu.SemaphoreType.DMA,
            pltpu.SemaphoreType.DMA,
        ],
        compiler_params=pltpu.CompilerParams(use_tc_tiling_on_sc=False),
    )
    def k(table_hbm, idx_hbm, out_hbm, idx_v, *bufs_and_sems):
        bufs = bufs_and_sems[:NBUF]
        sem_g, sem_w = bufs_and_sems[NBUF:]
        wid = lax.axis_index("s") * NC + lax.axis_index("c")
        base = wid * b_per_w
        pltpu.sync_copy(idx_hbm.at[pl.ds(base, b_per_w)], idx_v)

        def wait_one_write():
            pltpu.make_async_copy(
                bufs[0], out_hbm.at[pl.ds(base, NB)], sem_w
            ).wait()

        def outer(g, _):
            descs = []
            for b in range(NBUF):
                t = g * NBUF + b

                @pl.when(g >= 1)
                def _():
                    wait_one_write()

                for j in range(NB):
                    desc = pltpu.make_async_copy(
                        table_hbm.at[idx_v.at[t * NB + j]],
                        bufs[b].at[j],
                        sem_g,
                    )
                    desc.start()
                    descs.append(desc)
            for b in range(NBUF):
                t = g * NBUF + b
                for j in range(NB):
                    descs[b * NB + j].wait()
                pltpu.make_async_copy(
                    bufs[b], out_hbm.at[pl.ds(base + t * NB, NB)], sem_w
                ).start()
            return 0

        lax.fori_loop(0, n_outer, outer, 0)
        for _ in range(NBUF):
            wait_one_write()

    return k(table, locations)


def kernel(locations, table):
    batch, hist = locations.shape
    return _sc_gather(table, locations, batch, hist)
